# baseline (speedup 1.0000x reference)
import functools

import jax
import jax.numpy as jnp
from jax import lax
from jax.experimental import pallas as pl
from jax.experimental.pallas import tpu as pltpu
from jax.experimental.pallas import tpu_sc as plsc

NUM_USERS = 25000
NUM_ITEMS = 25000
N = NUM_USERS + NUM_ITEMS
NP = 50176
HALF = NP // 2
ACC_ROWS = 25216
D = 64
E = 800000
N_LAYERS = 3
B = 1024

EPT = E // 16
SUP = 2000
NSUP = EPT // SUP
NGRP = SUP // 16
CAP = SUP + 128
CH = 80
ZPT = ACC_ROWS // 16
WPT = HALF // 16

_mesh = plsc.VectorSubcoreMesh(core_axis_name="c", subcore_axis_name="s")
_sc_params = pltpu.CompilerParams(
    needs_layout_passes=False, use_tc_tiling_on_sc=False)


@functools.partial(
    pl.kernel,
    out_type=jax.ShapeDtypeStruct((NP, D), jnp.bfloat16),
    mesh=_mesh,
    scratch_types=[
        pltpu.VMEM((SUP,), jnp.int32),
        pltpu.VMEM((SUP,), jnp.int32),
        pltpu.VMEM((SUP,), jnp.float32),
        pltpu.VMEM((CAP,), jnp.int32),
        pltpu.VMEM((CAP,), jnp.int32),
        pltpu.VMEM((CAP,), jnp.float32),
        pltpu.VMEM((CH, D), jnp.bfloat16),
        pltpu.VMEM((CH, D), jnp.bfloat16),
        pltpu.VMEM((CH, D), jnp.float32),
        pltpu.VMEM((CH, D), jnp.float32),
        pltpu.VMEM((CH,), jnp.int32),
        pltpu.VMEM((CH,), jnp.int32),
        pltpu.VMEM_SHARED((ACC_ROWS, D), jnp.float32),
        pltpu.SemaphoreType.DMA,
        pltpu.SemaphoreType.DMA,
        pltpu.SemaphoreType.DMA,
        pltpu.SemaphoreType.DMA,
        pltpu.SemaphoreType.DMA,
    ],
    compiler_params=_sc_params,
)
def _layer(x_hbm, src_hbm, dst_hbm, w_hbm, y_hbm,
           srcb, dstb, wb, srcc, dlocc, wc, rowsa, rowsb, sbufa, sbufb,
           ixa, ixb, acc, seml, semga, semgb, semsa, semsb):
    sc = lax.axis_index("c")
    sub = lax.axis_index("s")
    half_base = sc * HALF
    zero16 = jnp.zeros((16,), jnp.float32)

    def zrow(i, _):
        for q in range(4):
            sbufa[i, pl.ds(q * 16, 16)] = zero16
        return 0
    with jax.named_scope("zero"):
        lax.fori_loop(0, CH, zrow, 0)
        zbase = sub * ZPT
        def zcopy(c, _):
            pltpu.sync_copy(sbufa.at[pl.ds(0, CH)],
                            acc.at[pl.ds(zbase + c * CH, CH)])
            return 0
        lax.fori_loop(0, ZPT // CH, zcopy, 0)
        pltpu.sync_copy(sbufa.at[pl.ds(0, ZPT % CH)],
                        acc.at[pl.ds(zbase + (ZPT // CH) * CH, ZPT % CH)])
    plsc.subcore_barrier()

    def sup_body(s, _):
        sbase = sub * EPT + s * SUP
        with jax.named_scope("sup_load"):
            l1 = pltpu.async_copy(src_hbm.at[pl.ds(sbase, SUP)], srcb, seml)
            l2 = pltpu.async_copy(dst_hbm.at[pl.ds(sbase, SUP)], dstb, seml)
            l3 = pltpu.async_copy(w_hbm.at[pl.ds(sbase, SUP)], wb, seml)
            l1.wait()
            l2.wait()
            l3.wait()

        def grp(i, cnt):
            sl = pl.ds(i * 16, 16)
            t = dstb[sl] - half_base
            ok = (t >= 0) & (t < HALF)
            plsc.store_compressed(srcc.at[pl.ds(cnt, 16)], srcb[sl], mask=ok)
            plsc.store_compressed(dlocc.at[pl.ds(cnt, 16)], t, mask=ok)
            plsc.store_compressed(wc.at[pl.ds(cnt, 16)], wb[sl], mask=ok)
            return cnt + jnp.sum(ok.astype(jnp.int32))
        with jax.named_scope("compact"):
            cnt = lax.fori_loop(0, NGRP, grp, jnp.int32(0))

        dummy16 = jnp.full((16,), HALF, jnp.int32)
        zrow16 = jnp.zeros((16,), jnp.int32)
        for k in range(CH // 16):
            srcc[pl.ds(cnt + k * 16, 16)] = zrow16
            dlocc[pl.ds(cnt + k * 16, 16)] = dummy16
            wc[pl.ds(cnt + k * 16, 16)] = zero16

        trip = lax.div(cnt + (CH - 1), jnp.int32(CH))
        pairs = lax.div(trip + 1, jnp.int32(2))

        def pair(p, _):
            off0 = p * (2 * CH)
            off1 = off0 + CH
            has1 = off1 < cnt
            da = pltpu.async_copy(
                x_hbm.at[srcc.at[pl.ds(off0, CH)]], rowsa, semga)
            @pl.when(has1)
            def _():
                pltpu.async_copy(
                    x_hbm.at[srcc.at[pl.ds(off1, CH)]], rowsb, semgb)
            da.wait()

            def scale(rows, sbuf, off):
                @plsc.parallel_loop(0, CH, unroll=2)
                def _(e):
                    w16 = plsc.load_gather(
                        wc, [jnp.full((16,), off + e, jnp.int32)])
                    for h in range(2):
                        ab = rows[e, pl.ds(h * 32, 32)]
                        a, b = plsc.unpack(
                            ab, format=plsc.PackFormat.INTERLEAVED)
                        sbuf[e, pl.ds(h * 32, 16)] = a * w16
                        sbuf[e, pl.ds(h * 32 + 16, 16)] = b * w16

            def stage_idx(ix, off):
                for k in range(CH // 16):
                    ix[pl.ds(k * 16, 16)] = dlocc[pl.ds(off + k * 16, 16)]

            scale(rowsa, sbufa, off0)
            stage_idx(ixa, off0)
            sa = pltpu.async_copy(sbufa, acc.at[ixa], semsa, add=True)

            @pl.when(has1)
            def _():
                pltpu.make_async_copy(
                    x_hbm.at[srcc.at[pl.ds(off1, CH)]], rowsb, semgb).wait()
                scale(rowsb, sbufb, off1)
                stage_idx(ixb, off1)
                pltpu.async_copy(sbufb, acc.at[ixb], semsb, add=True)

            sa.wait()
            @pl.when(has1)
            def _():
                pltpu.make_async_copy(sbufb, acc.at[ixb], semsb).wait()
            return 0
        with jax.named_scope("chunks"):
            lax.fori_loop(0, pairs, pair, 0)
        return 0
    lax.fori_loop(0, NSUP, sup_body, 0)
    plsc.subcore_barrier()

    with jax.named_scope("writeback"):
        wbase_l = sub * WPT
        wbase_g = half_base + wbase_l
        def pack_rows(nrows):
            def prow(i, _):
                for h in range(2):
                    a = sbufa[i, pl.ds(h * 32, 16)]
                    b = sbufa[i, pl.ds(h * 32 + 16, 16)]
                    rowsa[i, pl.ds(h * 32, 32)] = plsc.pack(
                        a, b, format=plsc.PackFormat.INTERLEAVED)
                return 0
            lax.fori_loop(0, nrows, prow, 0)
        def wchunk(c, _):
            pltpu.sync_copy(acc.at[pl.ds(wbase_l + c * CH, CH)],
                            sbufa.at[pl.ds(0, CH)])
            pack_rows(CH)
            pltpu.sync_copy(rowsa.at[pl.ds(0, CH)],
                            y_hbm.at[pl.ds(wbase_g + c * CH, CH)])
            return 0
        lax.fori_loop(0, WPT // CH, wchunk, 0)
        pltpu.sync_copy(acc.at[pl.ds(wbase_l + (WPT // CH) * CH, WPT % CH)],
                        sbufa.at[pl.ds(0, WPT % CH)])
        pack_rows(WPT % CH)
        pltpu.sync_copy(rowsa.at[pl.ds(0, WPT % CH)],
                        y_hbm.at[pl.ds(wbase_g + (WPT // CH) * CH, WPT % CH)])


@functools.partial(
    pl.kernel,
    out_type=jax.ShapeDtypeStruct((NP, D), jnp.bfloat16),
    mesh=_mesh,
    scratch_types=[
        pltpu.VMEM((CH, D), jnp.float32),
        pltpu.VMEM((CH, D), jnp.bfloat16),
        pltpu.SemaphoreType.DMA,
    ],
    compiler_params=_sc_params,
)
def _to_bf16(x_hbm, out_hbm, fbuf, bbuf, sem):
    wid = lax.axis_index("s") * 2 + lax.axis_index("c")
    rpt = NP // 32
    base = wid * rpt
    def conv(nrows, off):
        pltpu.sync_copy(x_hbm.at[pl.ds(off, nrows)], fbuf.at[pl.ds(0, nrows)])
        def prow(i, _):
            for h in range(2):
                a = fbuf[i, pl.ds(h * 32, 16)]
                b = fbuf[i, pl.ds(h * 32 + 16, 16)]
                bbuf[i, pl.ds(h * 32, 32)] = plsc.pack(
                    a, b, format=plsc.PackFormat.INTERLEAVED)
            return 0
        lax.fori_loop(0, nrows, prow, 0)
        pltpu.sync_copy(bbuf.at[pl.ds(0, nrows)],
                        out_hbm.at[pl.ds(off, nrows)])
    def cchunk(c, _):
        conv(CH, base + c * CH)
        return 0
    lax.fori_loop(0, rpt // CH, cchunk, 0)
    conv(rpt % CH, base + (rpt // CH) * CH)


@functools.partial(
    pl.kernel,
    out_type=jax.ShapeDtypeStruct((B, D), jnp.float32),
    mesh=_mesh,
    scratch_types=[
        pltpu.VMEM((B // 32,), jnp.int32),
        pltpu.VMEM((B // 32, D), jnp.float32),
        pltpu.SemaphoreType.DMA,
    ],
    compiler_params=_sc_params,
)
def _gather_users(s_hbm, users_hbm, out_hbm, idxv, rowsv, sem):
    wid = lax.axis_index("s") * 2 + lax.axis_index("c")
    base = wid * (B // 32)
    pltpu.sync_copy(users_hbm.at[pl.ds(base, B // 32)], idxv)
    pltpu.async_copy(s_hbm.at[idxv], rowsv, sem).wait()
    pltpu.sync_copy(rowsv, out_hbm.at[pl.ds(base, B // 32)])


def _sum4_body(a, b, c, d, o):
    af = a[...].astype(jnp.float32)
    bf = b[...].astype(jnp.float32)
    cf = c[...].astype(jnp.float32)
    df = d[...].astype(jnp.float32)
    o[...] = (af + bf) + (cf + df)


_sum4 = pl.pallas_call(
    _sum4_body,
    out_shape=jax.ShapeDtypeStruct((NP, D), jnp.float32),
    grid=(8,),
    in_specs=[pl.BlockSpec((NP // 8, D), lambda j: (j, 0))] * 4,
    out_specs=pl.BlockSpec((NP // 8, D), lambda j: (j, 0)),
)


UROW = 128


def _rating_body(u_ref, t_ref, o_ref):
    acc = lax.dot_general(u_ref[...], t_ref[...], (((1,), (1,)), ((), ())),
                          preferred_element_type=jnp.float32)
    o_ref[...] = jax.nn.sigmoid(acc * (1.0 / (N_LAYERS + 1) ** 2))


_rating = pl.pallas_call(
    _rating_body,
    out_shape=jax.ShapeDtypeStruct((B, NUM_ITEMS), jnp.float32),
    grid=(B // UROW,),
    in_specs=[
        pl.BlockSpec((UROW, D), lambda j: (j, 0)),
        pl.BlockSpec((NUM_ITEMS, D), lambda j: (0, 0)),
    ],
    out_specs=pl.BlockSpec((UROW, NUM_ITEMS), lambda j: (j, 0)),
)


def kernel(user_emb, item_emb, edge_index, edge_weight, users):
    x0 = jnp.concatenate(
        [user_emb, item_emb, jnp.zeros((NP - N, D), jnp.float32)], axis=0)
    src = edge_index[0]
    dst = edge_index[1]
    xs = [_to_bf16(x0)]
    for _ in range(N_LAYERS):
        xs.append(_layer(xs[-1], src, dst, edge_weight))
    s = _sum4(xs[0], xs[1], xs[2], xs[3])
    u = _gather_users(s, users)
    return _rating(u, s[NUM_USERS:N])


# speedup vs baseline: 1.0504x; 1.0504x over previous
import functools

import jax
import jax.numpy as jnp
from jax import lax
from jax.experimental import pallas as pl
from jax.experimental.pallas import tpu as pltpu
from jax.experimental.pallas import tpu_sc as plsc

NUM_USERS = 25000
NUM_ITEMS = 25000
N = NUM_USERS + NUM_ITEMS
NP = 50176
HALF = NP // 2
ACC_ROWS = 25216
D = 64
E = 800000
N_LAYERS = 3
B = 1024

EPT = E // 16
SUP = 2000
NSUP = EPT // SUP
NGRP = SUP // 16
CAP = SUP + 128
CH = 80
ZPT = ACC_ROWS // 16
WPT = HALF // 16

_mesh = plsc.VectorSubcoreMesh(core_axis_name="c", subcore_axis_name="s")
_sc_params = pltpu.CompilerParams(
    needs_layout_passes=False, use_tc_tiling_on_sc=False)


@functools.partial(
    pl.kernel,
    out_type=jax.ShapeDtypeStruct((NP, D), jnp.bfloat16),
    mesh=_mesh,
    scratch_types=[
        pltpu.VMEM((SUP,), jnp.int32),
        pltpu.VMEM((SUP,), jnp.int32),
        pltpu.VMEM((SUP,), jnp.float32),
        pltpu.VMEM((CAP,), jnp.int32),
        pltpu.VMEM((CAP,), jnp.int32),
        pltpu.VMEM((CAP,), jnp.float32),
        pltpu.VMEM((CH, D), jnp.bfloat16),
        pltpu.VMEM((CH, D), jnp.bfloat16),
        pltpu.VMEM((CH, D), jnp.float32),
        pltpu.VMEM((CH, D), jnp.float32),
        pltpu.VMEM((CH,), jnp.int32),
        pltpu.VMEM((CH,), jnp.int32),
        pltpu.VMEM_SHARED((ACC_ROWS, D), jnp.float32),
        pltpu.SemaphoreType.DMA,
        pltpu.SemaphoreType.DMA,
        pltpu.SemaphoreType.DMA,
        pltpu.SemaphoreType.DMA,
        pltpu.SemaphoreType.DMA,
    ],
    compiler_params=_sc_params,
)
def _layer(x_hbm, src_hbm, dst_hbm, w_hbm, y_hbm,
           srcb, dstb, wb, srcc, dlocc, wc, rowsa, rowsb, sbufa, sbufb,
           ixa, ixb, acc, seml, semga, semgb, semsa, semsb):
    sc = lax.axis_index("c")
    sub = lax.axis_index("s")
    half_base = sc * HALF
    zero16 = jnp.zeros((16,), jnp.float32)

    def zrow(i, _):
        for q in range(4):
            sbufa[i, pl.ds(q * 16, 16)] = zero16
        return 0
    with jax.named_scope("zero"):
        lax.fori_loop(0, CH, zrow, 0)
        zbase = sub * ZPT
        def zcopy(c, _):
            pltpu.sync_copy(sbufa.at[pl.ds(0, CH)],
                            acc.at[pl.ds(zbase + c * CH, CH)])
            return 0
        lax.fori_loop(0, ZPT // CH, zcopy, 0)
        pltpu.sync_copy(sbufa.at[pl.ds(0, ZPT % CH)],
                        acc.at[pl.ds(zbase + (ZPT // CH) * CH, ZPT % CH)])
    plsc.subcore_barrier()

    ebase = sub * EPT

    def issue_blk_loads(sbase):
        pltpu.async_copy(src_hbm.at[pl.ds(sbase, SUP)], srcb, seml)
        pltpu.async_copy(dst_hbm.at[pl.ds(sbase, SUP)], dstb, seml)
        pltpu.async_copy(w_hbm.at[pl.ds(sbase, SUP)], wb, seml)

    issue_blk_loads(ebase)

    def sup_body(s, _):
        sbase = ebase + s * SUP
        with jax.named_scope("sup_load"):
            pltpu.make_async_copy(
                src_hbm.at[pl.ds(sbase, SUP)], srcb, seml).wait()
            pltpu.make_async_copy(
                dst_hbm.at[pl.ds(sbase, SUP)], dstb, seml).wait()
            pltpu.make_async_copy(
                w_hbm.at[pl.ds(sbase, SUP)], wb, seml).wait()

        def grp(i, cnt):
            sl = pl.ds(i * 16, 16)
            t = dstb[sl] - half_base
            ok = (t >= 0) & (t < HALF)
            plsc.store_compressed(srcc.at[pl.ds(cnt, 16)], srcb[sl], mask=ok)
            plsc.store_compressed(dlocc.at[pl.ds(cnt, 16)], t, mask=ok)
            plsc.store_compressed(wc.at[pl.ds(cnt, 16)], wb[sl], mask=ok)
            return cnt + jnp.sum(ok.astype(jnp.int32))
        with jax.named_scope("compact"):
            cnt = lax.fori_loop(0, NGRP, grp, jnp.int32(0))

        dummy16 = jnp.full((16,), HALF, jnp.int32)
        zrow16 = jnp.zeros((16,), jnp.int32)
        for k in range(CH // 16):
            srcc[pl.ds(cnt + k * 16, 16)] = zrow16
            dlocc[pl.ds(cnt + k * 16, 16)] = dummy16
            wc[pl.ds(cnt + k * 16, 16)] = zero16

        trip = lax.div(cnt + (CH - 1), jnp.int32(CH))
        pairs = lax.div(trip + 1, jnp.int32(2))

        @pl.when(s + 1 < NSUP)
        def _():
            issue_blk_loads(sbase + SUP)

        def scale(rows, sbuf, off):
            @plsc.parallel_loop(0, CH, unroll=2)
            def _(e):
                w16 = plsc.load_gather(
                    wc, [jnp.full((16,), off + e, jnp.int32)])
                for h in range(2):
                    ab = rows[e, pl.ds(h * 32, 32)]
                    a, b = plsc.unpack(
                        ab, format=plsc.PackFormat.INTERLEAVED)
                    sbuf[e, pl.ds(h * 32, 16)] = a * w16
                    sbuf[e, pl.ds(h * 32 + 16, 16)] = b * w16

        def stage_idx(ix, off):
            for k in range(CH // 16):
                ix[pl.ds(k * 16, 16)] = dlocc[pl.ds(off + k * 16, 16)]

        @pl.when(trip > 0)
        def _():
            pltpu.async_copy(x_hbm.at[srcc.at[pl.ds(0, CH)]], rowsa, semga)
        @pl.when(trip > 1)
        def _():
            pltpu.async_copy(x_hbm.at[srcc.at[pl.ds(CH, CH)]], rowsb, semgb)

        def pair(p, _):
            off0 = p * (2 * CH)
            off1 = off0 + CH
            pltpu.make_async_copy(
                x_hbm.at[srcc.at[pl.ds(off0, CH)]], rowsa, semga).wait()
            @pl.when(p > 0)
            def _():
                pltpu.make_async_copy(sbufa, acc.at[ixa], semsa).wait()
            scale(rowsa, sbufa, off0)
            stage_idx(ixa, off0)
            pltpu.async_copy(sbufa, acc.at[ixa], semsa, add=True)
            @pl.when(2 * p + 2 < trip)
            def _():
                pltpu.async_copy(
                    x_hbm.at[srcc.at[pl.ds(off0 + 2 * CH, CH)]], rowsa, semga)

            @pl.when(2 * p + 1 < trip)
            def _():
                pltpu.make_async_copy(
                    x_hbm.at[srcc.at[pl.ds(off1, CH)]], rowsb, semgb).wait()
                @pl.when(p > 0)
                def _():
                    pltpu.make_async_copy(sbufb, acc.at[ixb], semsb).wait()
                scale(rowsb, sbufb, off1)
                stage_idx(ixb, off1)
                pltpu.async_copy(sbufb, acc.at[ixb], semsb, add=True)
                @pl.when(2 * p + 3 < trip)
                def _():
                    pltpu.async_copy(
                        x_hbm.at[srcc.at[pl.ds(off1 + 2 * CH, CH)]],
                        rowsb, semgb)
            return 0
        with jax.named_scope("chunks"):
            lax.fori_loop(0, pairs, pair, 0)
        @pl.when(trip > 0)
        def _():
            pltpu.make_async_copy(sbufa, acc.at[ixa], semsa).wait()
        @pl.when(trip > 1)
        def _():
            pltpu.make_async_copy(sbufb, acc.at[ixb], semsb).wait()
        return 0
    lax.fori_loop(0, NSUP, sup_body, 0)
    plsc.subcore_barrier()

    with jax.named_scope("writeback"):
        wbase_l = sub * WPT
        wbase_g = half_base + wbase_l
        def pack_rows(nrows):
            def prow(i, _):
                for h in range(2):
                    a = sbufa[i, pl.ds(h * 32, 16)]
                    b = sbufa[i, pl.ds(h * 32 + 16, 16)]
                    rowsa[i, pl.ds(h * 32, 32)] = plsc.pack(
                        a, b, format=plsc.PackFormat.INTERLEAVED)
                return 0
            lax.fori_loop(0, nrows, prow, 0)
        def wchunk(c, _):
            pltpu.sync_copy(acc.at[pl.ds(wbase_l + c * CH, CH)],
                            sbufa.at[pl.ds(0, CH)])
            pack_rows(CH)
            pltpu.sync_copy(rowsa.at[pl.ds(0, CH)],
                            y_hbm.at[pl.ds(wbase_g + c * CH, CH)])
            return 0
        lax.fori_loop(0, WPT // CH, wchunk, 0)
        pltpu.sync_copy(acc.at[pl.ds(wbase_l + (WPT // CH) * CH, WPT % CH)],
                        sbufa.at[pl.ds(0, WPT % CH)])
        pack_rows(WPT % CH)
        pltpu.sync_copy(rowsa.at[pl.ds(0, WPT % CH)],
                        y_hbm.at[pl.ds(wbase_g + (WPT // CH) * CH, WPT % CH)])


@functools.partial(
    pl.kernel,
    out_type=jax.ShapeDtypeStruct((NP, D), jnp.bfloat16),
    mesh=_mesh,
    scratch_types=[
        pltpu.VMEM((CH, D), jnp.float32),
        pltpu.VMEM((CH, D), jnp.bfloat16),
        pltpu.SemaphoreType.DMA,
    ],
    compiler_params=_sc_params,
)
def _to_bf16(x_hbm, out_hbm, fbuf, bbuf, sem):
    wid = lax.axis_index("s") * 2 + lax.axis_index("c")
    rpt = NP // 32
    base = wid * rpt
    def conv(nrows, off):
        pltpu.sync_copy(x_hbm.at[pl.ds(off, nrows)], fbuf.at[pl.ds(0, nrows)])
        def prow(i, _):
            for h in range(2):
                a = fbuf[i, pl.ds(h * 32, 16)]
                b = fbuf[i, pl.ds(h * 32 + 16, 16)]
                bbuf[i, pl.ds(h * 32, 32)] = plsc.pack(
                    a, b, format=plsc.PackFormat.INTERLEAVED)
            return 0
        lax.fori_loop(0, nrows, prow, 0)
        pltpu.sync_copy(bbuf.at[pl.ds(0, nrows)],
                        out_hbm.at[pl.ds(off, nrows)])
    def cchunk(c, _):
        conv(CH, base + c * CH)
        return 0
    lax.fori_loop(0, rpt // CH, cchunk, 0)
    conv(rpt % CH, base + (rpt // CH) * CH)


@functools.partial(
    pl.kernel,
    out_type=jax.ShapeDtypeStruct((B, D), jnp.float32),
    mesh=_mesh,
    scratch_types=[
        pltpu.VMEM((B // 32,), jnp.int32),
        pltpu.VMEM((B // 32, D), jnp.float32),
        pltpu.SemaphoreType.DMA,
    ],
    compiler_params=_sc_params,
)
def _gather_users(s_hbm, users_hbm, out_hbm, idxv, rowsv, sem):
    wid = lax.axis_index("s") * 2 + lax.axis_index("c")
    base = wid * (B // 32)
    pltpu.sync_copy(users_hbm.at[pl.ds(base, B // 32)], idxv)
    pltpu.async_copy(s_hbm.at[idxv], rowsv, sem).wait()
    pltpu.sync_copy(rowsv, out_hbm.at[pl.ds(base, B // 32)])


def _sum4_body(a, b, c, d, o):
    af = a[...].astype(jnp.float32)
    bf = b[...].astype(jnp.float32)
    cf = c[...].astype(jnp.float32)
    df = d[...].astype(jnp.float32)
    o[...] = (af + bf) + (cf + df)


_sum4 = pl.pallas_call(
    _sum4_body,
    out_shape=jax.ShapeDtypeStruct((NP, D), jnp.float32),
    grid=(8,),
    in_specs=[pl.BlockSpec((NP // 8, D), lambda j: (j, 0))] * 4,
    out_specs=pl.BlockSpec((NP // 8, D), lambda j: (j, 0)),
)


UROW = 128


def _rating_body(u_ref, t_ref, o_ref):
    acc = lax.dot_general(u_ref[...], t_ref[...], (((1,), (1,)), ((), ())),
                          preferred_element_type=jnp.float32)
    o_ref[...] = jax.nn.sigmoid(acc * (1.0 / (N_LAYERS + 1) ** 2))


_rating = pl.pallas_call(
    _rating_body,
    out_shape=jax.ShapeDtypeStruct((B, NUM_ITEMS), jnp.float32),
    grid=(B // UROW,),
    in_specs=[
        pl.BlockSpec((UROW, D), lambda j: (j, 0)),
        pl.BlockSpec((NUM_ITEMS, D), lambda j: (0, 0)),
    ],
    out_specs=pl.BlockSpec((UROW, NUM_ITEMS), lambda j: (j, 0)),
)


def kernel(user_emb, item_emb, edge_index, edge_weight, users):
    x0 = jnp.concatenate(
        [user_emb, item_emb, jnp.zeros((NP - N, D), jnp.float32)], axis=0)
    src = edge_index[0]
    dst = edge_index[1]
    xs = [_to_bf16(x0)]
    for _ in range(N_LAYERS):
        xs.append(_layer(xs[-1], src, dst, edge_weight))
    s = _sum4(xs[0], xs[1], xs[2], xs[3])
    u = _gather_users(s, users)
    return _rating(u, s[NUM_USERS:N])


# speedup vs baseline: 1.0511x; 1.0006x over previous
import functools

import jax
import jax.numpy as jnp
from jax import lax
from jax.experimental import pallas as pl
from jax.experimental.pallas import tpu as pltpu
from jax.experimental.pallas import tpu_sc as plsc

NUM_USERS = 25000
NUM_ITEMS = 25000
N = NUM_USERS + NUM_ITEMS
NP = 50176
HALF = NP // 2
ACC_ROWS = 25216
D = 64
E = 800000
N_LAYERS = 3
B = 1024

EPT = E // 16
SUP = 2000
NSUP = EPT // SUP
NGRP = SUP // 16
CAP = SUP + 128
CH = 80
ZPT = ACC_ROWS // 16
WPT = HALF // 16

_mesh = plsc.VectorSubcoreMesh(core_axis_name="c", subcore_axis_name="s")
_sc_params = pltpu.CompilerParams(
    needs_layout_passes=False, use_tc_tiling_on_sc=False)


@functools.partial(
    pl.kernel,
    out_type=jax.ShapeDtypeStruct((NP, D), jnp.bfloat16),
    mesh=_mesh,
    scratch_types=[
        pltpu.VMEM((SUP,), jnp.int32),
        pltpu.VMEM((SUP,), jnp.int32),
        pltpu.VMEM((SUP,), jnp.float32),
        pltpu.VMEM((CAP,), jnp.int32),
        pltpu.VMEM((CAP,), jnp.int32),
        pltpu.VMEM((CAP,), jnp.float32),
        pltpu.VMEM((CH, D), jnp.bfloat16),
        pltpu.VMEM((CH, D), jnp.bfloat16),
        pltpu.VMEM((CH, D), jnp.float32),
        pltpu.VMEM((CH, D), jnp.float32),
        pltpu.VMEM((CH,), jnp.int32),
        pltpu.VMEM((CH,), jnp.int32),
        pltpu.VMEM_SHARED((ACC_ROWS, D), jnp.float32),
        pltpu.SemaphoreType.DMA,
        pltpu.SemaphoreType.DMA,
        pltpu.SemaphoreType.DMA,
        pltpu.SemaphoreType.DMA,
        pltpu.SemaphoreType.DMA,
    ],
    compiler_params=_sc_params,
)
def _layer(x_hbm, src_hbm, dst_hbm, w_hbm, y_hbm,
           srcb, dstb, wb, srcc, dlocc, wc, rowsa, rowsb, sbufa, sbufb,
           ixa, ixb, acc, seml, semga, semgb, semsa, semsb):
    sc = lax.axis_index("c")
    sub = lax.axis_index("s")
    half_base = sc * HALF
    zero16 = jnp.zeros((16,), jnp.float32)

    def zrow(i, _):
        for q in range(4):
            sbufa[i, pl.ds(q * 16, 16)] = zero16
        return 0
    with jax.named_scope("zero"):
        lax.fori_loop(0, CH, zrow, 0)
        zbase = sub * ZPT
        def zcopy(c, _):
            pltpu.sync_copy(sbufa.at[pl.ds(0, CH)],
                            acc.at[pl.ds(zbase + c * CH, CH)])
            return 0
        lax.fori_loop(0, ZPT // CH, zcopy, 0)
        pltpu.sync_copy(sbufa.at[pl.ds(0, ZPT % CH)],
                        acc.at[pl.ds(zbase + (ZPT // CH) * CH, ZPT % CH)])
    plsc.subcore_barrier()

    ebase = sub * EPT

    def issue_blk_loads(sbase):
        pltpu.async_copy(src_hbm.at[pl.ds(sbase, SUP)], srcb, seml)
        pltpu.async_copy(dst_hbm.at[pl.ds(sbase, SUP)], dstb, seml)
        pltpu.async_copy(w_hbm.at[pl.ds(sbase, SUP)], wb, seml)

    issue_blk_loads(ebase)

    def sup_body(s, _):
        sbase = ebase + s * SUP
        with jax.named_scope("sup_load"):
            pltpu.make_async_copy(
                src_hbm.at[pl.ds(sbase, SUP)], srcb, seml).wait()
            pltpu.make_async_copy(
                dst_hbm.at[pl.ds(sbase, SUP)], dstb, seml).wait()
            pltpu.make_async_copy(
                w_hbm.at[pl.ds(sbase, SUP)], wb, seml).wait()

        def grp(i, cnt):
            sl = pl.ds(i * 16, 16)
            t = dstb[sl] - half_base
            ok = (t >= 0) & (t < HALF)
            plsc.store_compressed(srcc.at[pl.ds(cnt, 16)], srcb[sl], mask=ok)
            plsc.store_compressed(dlocc.at[pl.ds(cnt, 16)], t, mask=ok)
            plsc.store_compressed(wc.at[pl.ds(cnt, 16)], wb[sl], mask=ok)
            return cnt + jnp.sum(ok.astype(jnp.int32))
        with jax.named_scope("compact"):
            cnt = lax.fori_loop(0, NGRP, grp, jnp.int32(0))

        dummy16 = jnp.full((16,), HALF, jnp.int32)
        zrow16 = jnp.zeros((16,), jnp.int32)
        for k in range(CH // 16):
            srcc[pl.ds(cnt + k * 16, 16)] = zrow16
            dlocc[pl.ds(cnt + k * 16, 16)] = dummy16
            wc[pl.ds(cnt + k * 16, 16)] = zero16

        trip = lax.div(cnt + (CH - 1), jnp.int32(CH))
        pairs = lax.div(trip + 1, jnp.int32(2))

        @pl.when(s + 1 < NSUP)
        def _():
            issue_blk_loads(sbase + SUP)

        def scale(rows, sbuf, off):
            @plsc.parallel_loop(0, CH, unroll=2)
            def _(e):
                w16 = plsc.load_gather(
                    wc, [jnp.full((16,), off + e, jnp.int32)])
                for h in range(2):
                    ab = rows[e, pl.ds(h * 32, 32)]
                    a, b = plsc.unpack(
                        ab, format=plsc.PackFormat.INTERLEAVED)
                    sbuf[e, pl.ds(h * 32, 16)] = a * w16
                    sbuf[e, pl.ds(h * 32 + 16, 16)] = b * w16

        def stage_idx(ix, off):
            for k in range(CH // 16):
                ix[pl.ds(k * 16, 16)] = dlocc[pl.ds(off + k * 16, 16)]

        HC = CH // 2

        def issue_gather(buf, sem, off):
            pltpu.async_copy(
                x_hbm.at[srcc.at[pl.ds(off, HC)]], buf.at[pl.ds(0, HC)], sem)
            pltpu.async_copy(
                x_hbm.at[srcc.at[pl.ds(off + HC, HC)]],
                buf.at[pl.ds(HC, HC)], sem)

        def wait_gather(buf, sem, off):
            pltpu.make_async_copy(
                x_hbm.at[srcc.at[pl.ds(off, HC)]],
                buf.at[pl.ds(0, HC)], sem).wait()
            pltpu.make_async_copy(
                x_hbm.at[srcc.at[pl.ds(off + HC, HC)]],
                buf.at[pl.ds(HC, HC)], sem).wait()

        @pl.when(trip > 0)
        def _():
            issue_gather(rowsa, semga, 0)
        @pl.when(trip > 1)
        def _():
            issue_gather(rowsb, semgb, CH)

        def pair(p, _):
            off0 = p * (2 * CH)
            off1 = off0 + CH
            wait_gather(rowsa, semga, off0)
            @pl.when(p > 0)
            def _():
                pltpu.make_async_copy(sbufa, acc.at[ixa], semsa).wait()
            scale(rowsa, sbufa, off0)
            stage_idx(ixa, off0)
            pltpu.async_copy(sbufa, acc.at[ixa], semsa, add=True)
            @pl.when(2 * p + 2 < trip)
            def _():
                issue_gather(rowsa, semga, off0 + 2 * CH)

            @pl.when(2 * p + 1 < trip)
            def _():
                wait_gather(rowsb, semgb, off1)
                @pl.when(p > 0)
                def _():
                    pltpu.make_async_copy(sbufb, acc.at[ixb], semsb).wait()
                scale(rowsb, sbufb, off1)
                stage_idx(ixb, off1)
                pltpu.async_copy(sbufb, acc.at[ixb], semsb, add=True)
                @pl.when(2 * p + 3 < trip)
                def _():
                    issue_gather(rowsb, semgb, off1 + 2 * CH)
            return 0
        with jax.named_scope("chunks"):
            lax.fori_loop(0, pairs, pair, 0)
        @pl.when(trip > 0)
        def _():
            pltpu.make_async_copy(sbufa, acc.at[ixa], semsa).wait()
        @pl.when(trip > 1)
        def _():
            pltpu.make_async_copy(sbufb, acc.at[ixb], semsb).wait()
        return 0
    lax.fori_loop(0, NSUP, sup_body, 0)
    plsc.subcore_barrier()

    with jax.named_scope("writeback"):
        wbase_l = sub * WPT
        wbase_g = half_base + wbase_l
        def pack_rows(nrows):
            def prow(i, _):
                for h in range(2):
                    a = sbufa[i, pl.ds(h * 32, 16)]
                    b = sbufa[i, pl.ds(h * 32 + 16, 16)]
                    rowsa[i, pl.ds(h * 32, 32)] = plsc.pack(
                        a, b, format=plsc.PackFormat.INTERLEAVED)
                return 0
            lax.fori_loop(0, nrows, prow, 0)
        def wchunk(c, _):
            pltpu.sync_copy(acc.at[pl.ds(wbase_l + c * CH, CH)],
                            sbufa.at[pl.ds(0, CH)])
            pack_rows(CH)
            pltpu.sync_copy(rowsa.at[pl.ds(0, CH)],
                            y_hbm.at[pl.ds(wbase_g + c * CH, CH)])
            return 0
        lax.fori_loop(0, WPT // CH, wchunk, 0)
        pltpu.sync_copy(acc.at[pl.ds(wbase_l + (WPT // CH) * CH, WPT % CH)],
                        sbufa.at[pl.ds(0, WPT % CH)])
        pack_rows(WPT % CH)
        pltpu.sync_copy(rowsa.at[pl.ds(0, WPT % CH)],
                        y_hbm.at[pl.ds(wbase_g + (WPT // CH) * CH, WPT % CH)])


@functools.partial(
    pl.kernel,
    out_type=jax.ShapeDtypeStruct((NP, D), jnp.bfloat16),
    mesh=_mesh,
    scratch_types=[
        pltpu.VMEM((CH, D), jnp.float32),
        pltpu.VMEM((CH, D), jnp.bfloat16),
        pltpu.SemaphoreType.DMA,
    ],
    compiler_params=_sc_params,
)
def _to_bf16(x_hbm, out_hbm, fbuf, bbuf, sem):
    wid = lax.axis_index("s") * 2 + lax.axis_index("c")
    rpt = NP // 32
    base = wid * rpt
    def conv(nrows, off):
        pltpu.sync_copy(x_hbm.at[pl.ds(off, nrows)], fbuf.at[pl.ds(0, nrows)])
        def prow(i, _):
            for h in range(2):
                a = fbuf[i, pl.ds(h * 32, 16)]
                b = fbuf[i, pl.ds(h * 32 + 16, 16)]
                bbuf[i, pl.ds(h * 32, 32)] = plsc.pack(
                    a, b, format=plsc.PackFormat.INTERLEAVED)
            return 0
        lax.fori_loop(0, nrows, prow, 0)
        pltpu.sync_copy(bbuf.at[pl.ds(0, nrows)],
                        out_hbm.at[pl.ds(off, nrows)])
    def cchunk(c, _):
        conv(CH, base + c * CH)
        return 0
    lax.fori_loop(0, rpt // CH, cchunk, 0)
    conv(rpt % CH, base + (rpt // CH) * CH)


@functools.partial(
    pl.kernel,
    out_type=jax.ShapeDtypeStruct((B, D), jnp.float32),
    mesh=_mesh,
    scratch_types=[
        pltpu.VMEM((B // 32,), jnp.int32),
        pltpu.VMEM((B // 32, D), jnp.float32),
        pltpu.SemaphoreType.DMA,
    ],
    compiler_params=_sc_params,
)
def _gather_users(s_hbm, users_hbm, out_hbm, idxv, rowsv, sem):
    wid = lax.axis_index("s") * 2 + lax.axis_index("c")
    base = wid * (B // 32)
    pltpu.sync_copy(users_hbm.at[pl.ds(base, B // 32)], idxv)
    pltpu.async_copy(s_hbm.at[idxv], rowsv, sem).wait()
    pltpu.sync_copy(rowsv, out_hbm.at[pl.ds(base, B // 32)])


def _sum4_body(a, b, c, d, o):
    af = a[...].astype(jnp.float32)
    bf = b[...].astype(jnp.float32)
    cf = c[...].astype(jnp.float32)
    df = d[...].astype(jnp.float32)
    o[...] = (af + bf) + (cf + df)


_sum4 = pl.pallas_call(
    _sum4_body,
    out_shape=jax.ShapeDtypeStruct((NP, D), jnp.float32),
    grid=(8,),
    in_specs=[pl.BlockSpec((NP // 8, D), lambda j: (j, 0))] * 4,
    out_specs=pl.BlockSpec((NP // 8, D), lambda j: (j, 0)),
)


UROW = 128


def _rating_body(u_ref, t_ref, o_ref):
    acc = lax.dot_general(u_ref[...], t_ref[...], (((1,), (1,)), ((), ())),
                          preferred_element_type=jnp.float32)
    o_ref[...] = jax.nn.sigmoid(acc * (1.0 / (N_LAYERS + 1) ** 2))


_rating = pl.pallas_call(
    _rating_body,
    out_shape=jax.ShapeDtypeStruct((B, NUM_ITEMS), jnp.float32),
    grid=(B // UROW,),
    in_specs=[
        pl.BlockSpec((UROW, D), lambda j: (j, 0)),
        pl.BlockSpec((NUM_ITEMS, D), lambda j: (0, 0)),
    ],
    out_specs=pl.BlockSpec((UROW, NUM_ITEMS), lambda j: (j, 0)),
)


def kernel(user_emb, item_emb, edge_index, edge_weight, users):
    x0 = jnp.concatenate(
        [user_emb, item_emb, jnp.zeros((NP - N, D), jnp.float32)], axis=0)
    src = edge_index[0]
    dst = edge_index[1]
    xs = [_to_bf16(x0)]
    for _ in range(N_LAYERS):
        xs.append(_layer(xs[-1], src, dst, edge_weight))
    s = _sum4(xs[0], xs[1], xs[2], xs[3])
    u = _gather_users(s, users)
    return _rating(u, s[NUM_USERS:N])


# speedup vs baseline: 1.0672x; 1.0153x over previous
import functools

import jax
import jax.numpy as jnp
from jax import lax
from jax.experimental import pallas as pl
from jax.experimental.pallas import tpu as pltpu
from jax.experimental.pallas import tpu_sc as plsc

NUM_USERS = 25000
NUM_ITEMS = 25000
N = NUM_USERS + NUM_ITEMS
NP = 50176
HALF = NP // 2
ACC_ROWS = 25216
D = 64
E = 800000
N_LAYERS = 3
B = 1024

EPT = E // 16
SUP = 2000
NSUP = EPT // SUP
NGRP = SUP // 16
CAP = SUP + 128
CH = 80
ZPT = ACC_ROWS // 16
WPT = HALF // 16

_mesh = plsc.VectorSubcoreMesh(core_axis_name="c", subcore_axis_name="s")
_sc_params = pltpu.CompilerParams(
    needs_layout_passes=False, use_tc_tiling_on_sc=False)


@functools.partial(
    pl.kernel,
    out_type=jax.ShapeDtypeStruct((NP, D), jnp.bfloat16),
    mesh=_mesh,
    scratch_types=[
        pltpu.VMEM((SUP,), jnp.int32),
        pltpu.VMEM((SUP,), jnp.int32),
        pltpu.VMEM((SUP,), jnp.float32),
        pltpu.VMEM((CAP,), jnp.int32),
        pltpu.VMEM((CAP,), jnp.int32),
        pltpu.VMEM((CAP,), jnp.float32),
        pltpu.VMEM((CH, D), jnp.bfloat16),
        pltpu.VMEM((CH, D), jnp.bfloat16),
        pltpu.VMEM((CH, D), jnp.float32),
        pltpu.VMEM((CH, D), jnp.float32),
        pltpu.VMEM((CH,), jnp.int32),
        pltpu.VMEM((CH,), jnp.int32),
        pltpu.VMEM_SHARED((ACC_ROWS, D), jnp.float32),
        pltpu.SemaphoreType.DMA,
        pltpu.SemaphoreType.DMA,
        pltpu.SemaphoreType.DMA,
        pltpu.SemaphoreType.DMA,
        pltpu.SemaphoreType.DMA,
    ],
    compiler_params=_sc_params,
)
def _layer(x_hbm, src_hbm, dst_hbm, w_hbm, y_hbm,
           srcb, dstb, wb, srcc, dlocc, wc, rowsa, rowsb, sbufa, sbufb,
           ixa, ixb, acc, seml, semga, semgb, semsa, semsb):
    sc = lax.axis_index("c")
    sub = lax.axis_index("s")
    half_base = sc * HALF
    zero16 = jnp.zeros((16,), jnp.float32)

    WC = 56
    def zrow(i, _):
        for q in range(4):
            sbufa[i, pl.ds(q * 16, 16)] = zero16
        return 0
    with jax.named_scope("zero"):
        lax.fori_loop(0, WC, zrow, 0)
        zbase = sub * ZPT
        def zissue(c, _):
            pltpu.async_copy(sbufa.at[pl.ds(0, WC)],
                             acc.at[pl.ds(zbase + c * WC, WC)], semsa)
            return 0
        lax.fori_loop(0, ZPT // WC, zissue, 0)
        pltpu.sync_copy(sbufa.at[pl.ds(0, ZPT % WC)],
                        acc.at[pl.ds(zbase + (ZPT // WC) * WC, ZPT % WC)])
        def zdrain(c, _):
            pltpu.make_async_copy(
                sbufa.at[pl.ds(0, WC)],
                acc.at[pl.ds(zbase + c * WC, WC)], semsa).wait()
            return 0
        lax.fori_loop(0, ZPT // WC, zdrain, 0)
    plsc.subcore_barrier()

    ebase = sub * EPT

    def issue_blk_loads(sbase):
        pltpu.async_copy(src_hbm.at[pl.ds(sbase, SUP)], srcb, seml)
        pltpu.async_copy(dst_hbm.at[pl.ds(sbase, SUP)], dstb, seml)
        pltpu.async_copy(w_hbm.at[pl.ds(sbase, SUP)], wb, seml)

    issue_blk_loads(ebase)

    def sup_body(s, _):
        sbase = ebase + s * SUP
        with jax.named_scope("sup_load"):
            pltpu.make_async_copy(
                src_hbm.at[pl.ds(sbase, SUP)], srcb, seml).wait()
            pltpu.make_async_copy(
                dst_hbm.at[pl.ds(sbase, SUP)], dstb, seml).wait()
            pltpu.make_async_copy(
                w_hbm.at[pl.ds(sbase, SUP)], wb, seml).wait()

        def grp(i, cnt):
            sl = pl.ds(i * 16, 16)
            t = dstb[sl] - half_base
            ok = (t >= 0) & (t < HALF)
            plsc.store_compressed(srcc.at[pl.ds(cnt, 16)], srcb[sl], mask=ok)
            plsc.store_compressed(dlocc.at[pl.ds(cnt, 16)], t, mask=ok)
            plsc.store_compressed(wc.at[pl.ds(cnt, 16)], wb[sl], mask=ok)
            return cnt + jnp.sum(ok.astype(jnp.int32))
        with jax.named_scope("compact"):
            cnt = lax.fori_loop(0, NGRP, grp, jnp.int32(0))

        dummy16 = jnp.full((16,), HALF, jnp.int32)
        zrow16 = jnp.zeros((16,), jnp.int32)
        for k in range(CH // 16):
            srcc[pl.ds(cnt + k * 16, 16)] = zrow16
            dlocc[pl.ds(cnt + k * 16, 16)] = dummy16
            wc[pl.ds(cnt + k * 16, 16)] = zero16

        trip = lax.div(cnt + (CH - 1), jnp.int32(CH))
        pairs = lax.div(trip + 1, jnp.int32(2))

        @pl.when(s + 1 < NSUP)
        def _():
            issue_blk_loads(sbase + SUP)

        def scale(rows, sbuf, off):
            @plsc.parallel_loop(0, CH, unroll=2)
            def _(e):
                w16 = plsc.load_gather(
                    wc, [jnp.full((16,), off + e, jnp.int32)])
                for h in range(2):
                    ab = rows[e, pl.ds(h * 32, 32)]
                    a, b = plsc.unpack(
                        ab, format=plsc.PackFormat.INTERLEAVED)
                    sbuf[e, pl.ds(h * 32, 16)] = a * w16
                    sbuf[e, pl.ds(h * 32 + 16, 16)] = b * w16

        def stage_idx(ix, off):
            for k in range(CH // 16):
                ix[pl.ds(k * 16, 16)] = dlocc[pl.ds(off + k * 16, 16)]

        HC = CH // 2

        def issue_gather(buf, sem, off):
            pltpu.async_copy(
                x_hbm.at[srcc.at[pl.ds(off, HC)]], buf.at[pl.ds(0, HC)], sem)
            pltpu.async_copy(
                x_hbm.at[srcc.at[pl.ds(off + HC, HC)]],
                buf.at[pl.ds(HC, HC)], sem)

        def wait_gather(buf, sem, off):
            pltpu.make_async_copy(
                x_hbm.at[srcc.at[pl.ds(off, HC)]],
                buf.at[pl.ds(0, HC)], sem).wait()
            pltpu.make_async_copy(
                x_hbm.at[srcc.at[pl.ds(off + HC, HC)]],
                buf.at[pl.ds(HC, HC)], sem).wait()

        @pl.when(trip > 0)
        def _():
            issue_gather(rowsa, semga, 0)
        @pl.when(trip > 1)
        def _():
            issue_gather(rowsb, semgb, CH)

        def pair(p, _):
            off0 = p * (2 * CH)
            off1 = off0 + CH
            wait_gather(rowsa, semga, off0)
            @pl.when(p > 0)
            def _():
                pltpu.make_async_copy(sbufa, acc.at[ixa], semsa).wait()
            scale(rowsa, sbufa, off0)
            stage_idx(ixa, off0)
            pltpu.async_copy(sbufa, acc.at[ixa], semsa, add=True)
            @pl.when(2 * p + 2 < trip)
            def _():
                issue_gather(rowsa, semga, off0 + 2 * CH)

            @pl.when(2 * p + 1 < trip)
            def _():
                wait_gather(rowsb, semgb, off1)
                @pl.when(p > 0)
                def _():
                    pltpu.make_async_copy(sbufb, acc.at[ixb], semsb).wait()
                scale(rowsb, sbufb, off1)
                stage_idx(ixb, off1)
                pltpu.async_copy(sbufb, acc.at[ixb], semsb, add=True)
                @pl.when(2 * p + 3 < trip)
                def _():
                    issue_gather(rowsb, semgb, off1 + 2 * CH)
            return 0
        with jax.named_scope("chunks"):
            lax.fori_loop(0, pairs, pair, 0)
        @pl.when(trip > 0)
        def _():
            pltpu.make_async_copy(sbufa, acc.at[ixa], semsa).wait()
        @pl.when(trip > 1)
        def _():
            pltpu.make_async_copy(sbufb, acc.at[ixb], semsb).wait()
        return 0
    lax.fori_loop(0, NSUP, sup_body, 0)
    plsc.subcore_barrier()

    with jax.named_scope("writeback"):
        wbase_l = sub * WPT
        wbase_g = half_base + wbase_l
        NWCH = WPT // WC
        def rd(c, sbuf, sem):
            return pltpu.make_async_copy(
                acc.at[pl.ds(wbase_l + c * WC, WC)],
                sbuf.at[pl.ds(0, WC)], sem)
        def wr(c, rows, sem):
            return pltpu.make_async_copy(
                rows.at[pl.ds(0, WC)],
                y_hbm.at[pl.ds(wbase_g + c * WC, WC)], sem)
        def pack_rows(sbuf, rows):
            def prow(i, _):
                for h in range(2):
                    a = sbuf[i, pl.ds(h * 32, 16)]
                    b = sbuf[i, pl.ds(h * 32 + 16, 16)]
                    rows[i, pl.ds(h * 32, 32)] = plsc.pack(
                        a, b, format=plsc.PackFormat.INTERLEAVED)
                return 0
            lax.fori_loop(0, WC, prow, 0)
        rd(0, sbufa, semga).start()
        rd(1, sbufb, semgb).start()
        def wpair(p, _):
            c0 = 2 * p
            c1 = c0 + 1
            rd(c0, sbufa, semga).wait()
            @pl.when(p > 0)
            def _():
                wr(c0 - 2, rowsa, semsa).wait()
            pack_rows(sbufa, rowsa)
            wr(c0, rowsa, semsa).start()
            @pl.when(c0 + 2 < NWCH)
            def _():
                rd(c0 + 2, sbufa, semga).start()
            rd(c1, sbufb, semgb).wait()
            @pl.when(p > 0)
            def _():
                wr(c1 - 2, rowsb, semsb).wait()
            pack_rows(sbufb, rowsb)
            wr(c1, rowsb, semsb).start()
            @pl.when(c1 + 2 < NWCH)
            def _():
                rd(c1 + 2, sbufb, semgb).start()
            return 0
        lax.fori_loop(0, NWCH // 2, wpair, 0)
        wr(NWCH - 2, rowsa, semsa).wait()
        wr(NWCH - 1, rowsb, semsb).wait()


@functools.partial(
    pl.kernel,
    out_type=jax.ShapeDtypeStruct((NP, D), jnp.bfloat16),
    mesh=_mesh,
    scratch_types=[
        pltpu.VMEM((CH, D), jnp.float32),
        pltpu.VMEM((CH, D), jnp.bfloat16),
        pltpu.SemaphoreType.DMA,
    ],
    compiler_params=_sc_params,
)
def _to_bf16(x_hbm, out_hbm, fbuf, bbuf, sem):
    wid = lax.axis_index("s") * 2 + lax.axis_index("c")
    rpt = NP // 32
    base = wid * rpt
    def conv(nrows, off):
        pltpu.sync_copy(x_hbm.at[pl.ds(off, nrows)], fbuf.at[pl.ds(0, nrows)])
        def prow(i, _):
            for h in range(2):
                a = fbuf[i, pl.ds(h * 32, 16)]
                b = fbuf[i, pl.ds(h * 32 + 16, 16)]
                bbuf[i, pl.ds(h * 32, 32)] = plsc.pack(
                    a, b, format=plsc.PackFormat.INTERLEAVED)
            return 0
        lax.fori_loop(0, nrows, prow, 0)
        pltpu.sync_copy(bbuf.at[pl.ds(0, nrows)],
                        out_hbm.at[pl.ds(off, nrows)])
    def cchunk(c, _):
        conv(CH, base + c * CH)
        return 0
    lax.fori_loop(0, rpt // CH, cchunk, 0)
    conv(rpt % CH, base + (rpt // CH) * CH)


@functools.partial(
    pl.kernel,
    out_type=jax.ShapeDtypeStruct((B, D), jnp.float32),
    mesh=_mesh,
    scratch_types=[
        pltpu.VMEM((B // 32,), jnp.int32),
        pltpu.VMEM((B // 32, D), jnp.float32),
        pltpu.SemaphoreType.DMA,
    ],
    compiler_params=_sc_params,
)
def _gather_users(s_hbm, users_hbm, out_hbm, idxv, rowsv, sem):
    wid = lax.axis_index("s") * 2 + lax.axis_index("c")
    base = wid * (B // 32)
    pltpu.sync_copy(users_hbm.at[pl.ds(base, B // 32)], idxv)
    pltpu.async_copy(s_hbm.at[idxv], rowsv, sem).wait()
    pltpu.sync_copy(rowsv, out_hbm.at[pl.ds(base, B // 32)])


def _sum4_body(a, b, c, d, o):
    af = a[...].astype(jnp.float32)
    bf = b[...].astype(jnp.float32)
    cf = c[...].astype(jnp.float32)
    df = d[...].astype(jnp.float32)
    o[...] = (af + bf) + (cf + df)


_sum4 = pl.pallas_call(
    _sum4_body,
    out_shape=jax.ShapeDtypeStruct((NP, D), jnp.float32),
    grid=(8,),
    in_specs=[pl.BlockSpec((NP // 8, D), lambda j: (j, 0))] * 4,
    out_specs=pl.BlockSpec((NP // 8, D), lambda j: (j, 0)),
)


UROW = 128


def _rating_body(u_ref, t_ref, o_ref):
    acc = lax.dot_general(u_ref[...], t_ref[...], (((1,), (1,)), ((), ())),
                          preferred_element_type=jnp.float32)
    o_ref[...] = jax.nn.sigmoid(acc * (1.0 / (N_LAYERS + 1) ** 2))


_rating = pl.pallas_call(
    _rating_body,
    out_shape=jax.ShapeDtypeStruct((B, NUM_ITEMS), jnp.float32),
    grid=(B // UROW,),
    in_specs=[
        pl.BlockSpec((UROW, D), lambda j: (j, 0)),
        pl.BlockSpec((NUM_ITEMS, D), lambda j: (0, 0)),
    ],
    out_specs=pl.BlockSpec((UROW, NUM_ITEMS), lambda j: (j, 0)),
)


def kernel(user_emb, item_emb, edge_index, edge_weight, users):
    x0 = jnp.concatenate(
        [user_emb, item_emb, jnp.zeros((NP - N, D), jnp.float32)], axis=0)
    src = edge_index[0]
    dst = edge_index[1]
    xs = [_to_bf16(x0)]
    for _ in range(N_LAYERS):
        xs.append(_layer(xs[-1], src, dst, edge_weight))
    s = _sum4(xs[0], xs[1], xs[2], xs[3])
    u = _gather_users(s, users)
    return _rating(u, s[NUM_USERS:N])
